# initial kernel scaffold (unmeasured)
import jax
import jax.numpy as jnp
from jax import lax
from jax.experimental import pallas as pl
from jax.experimental.pallas import tpu as pltpu

N_DEV = 8


def kernel(x, k, Wp):
    B, S, C = x.shape
    T = k.shape[0]
    _, P = Wp.shape
    R = B * S
    CH = R // N_DEV

    def body(x_ref, k_ref, wp_ref, out_ref,
             pr_ref, ag_ref, rs_comm,
             rs_send, rs_recv, ag_send, ag_recv):
        d = lax.axis_index("i")
        left = (d - 1 + N_DEV) % N_DEV
        right = (d + 1) % N_DEV

        bar = pltpu.get_barrier_semaphore()
        pl.semaphore_signal(bar, inc=1, device_id=(left,),
                            device_id_type=pl.DeviceIdType.MESH)
        pl.semaphore_signal(bar, inc=1, device_id=(right,),
                            device_id_type=pl.DeviceIdType.MESH)
        pl.semaphore_wait(bar, 2)

        xv = x_ref[...]
        conv = xv * k_ref[T - 1, :]
        for t in range(T - 1):
            sh = T - 1 - t
            shifted = jnp.concatenate(
                [jnp.zeros((B, sh, C), jnp.float32), xv[:, :S - sh, :]],
                axis=1)
            conv = conv + shifted * k_ref[t, :]
        a = conv / (1.0 + jnp.exp(-conv))
        p = jnp.dot(a.reshape(R, C), wp_ref[...],
                    preferred_element_type=jnp.float32)

        for s in range(N_DEV):
            row = ((d - s + N_DEV) % N_DEV) * CH
            pr_ref[s, :, :] = lax.dynamic_slice(p, (row, 0), (CH, P))

        for h in range(N_DEV - 1):
            rdma = pltpu.make_async_remote_copy(
                src_ref=pr_ref.at[h],
                dst_ref=rs_comm.at[h],
                send_sem=rs_send.at[h],
                recv_sem=rs_recv.at[h],
                device_id=(right,),
                device_id_type=pl.DeviceIdType.MESH,
            )
            rdma.start()
            rdma.wait()
            pr_ref[h + 1, :, :] = pr_ref[h + 1, :, :] + rs_comm[h, :, :]

        ag_ref[0, :, :] = pr_ref[N_DEV - 1, :, :]
        for h in range(N_DEV - 1):
            rdma = pltpu.make_async_remote_copy(
                src_ref=ag_ref.at[h],
                dst_ref=ag_ref.at[h + 1],
                send_sem=ag_send.at[h],
                recv_sem=ag_recv.at[h],
                device_id=(right,),
                device_id_type=pl.DeviceIdType.MESH,
            )
            rdma.start()
            rdma.wait()

        for a_slot in range(N_DEV):
            c = (d + 1 - a_slot + N_DEV) % N_DEV
            out_ref[pl.ds(c * CH, CH), :] = ag_ref[a_slot, :, :]

    out = pl.pallas_call(
        body,
        out_shape=jax.ShapeDtypeStruct((R, P), jnp.float32),
        in_specs=[pl.BlockSpec(memory_space=pltpu.VMEM)] * 3,
        out_specs=pl.BlockSpec(memory_space=pltpu.VMEM),
        scratch_shapes=[
            pltpu.VMEM((N_DEV, CH, P), jnp.float32),
            pltpu.VMEM((N_DEV, CH, P), jnp.float32),
            pltpu.VMEM((N_DEV - 1, CH, P), jnp.float32),
            pltpu.SemaphoreType.DMA((N_DEV - 1,)),
            pltpu.SemaphoreType.DMA((N_DEV - 1,)),
            pltpu.SemaphoreType.DMA((N_DEV - 1,)),
            pltpu.SemaphoreType.DMA((N_DEV - 1,)),
        ],
        compiler_params=pltpu.CompilerParams(collective_id=0),
    )(x, k, Wp)
    return out.reshape(B, S, P)


# baseline (device time: 207529 ns/iter reference)
import jax
import jax.numpy as jnp
from jax import lax
from jax.experimental import pallas as pl
from jax.experimental.pallas import tpu as pltpu

N_DEV = 8


def kernel(x, k, Wp):
    B, S, C = x.shape
    T = k.shape[0]
    _, P = Wp.shape
    R = B * S
    CH = R // N_DEV

    def body(x_ref, k_ref, wp_ref, out_ref,
             p_ref, pr_ref, ag_ref, rs_comm,
             rs_send, rs_recv, ag_send, ag_recv):
        d = lax.axis_index("i")
        left = (d - 1 + N_DEV) % N_DEV
        right = (d + 1) % N_DEV

        bar = pltpu.get_barrier_semaphore()
        pl.semaphore_signal(bar, inc=1, device_id=(left,),
                            device_id_type=pl.DeviceIdType.MESH)
        pl.semaphore_signal(bar, inc=1, device_id=(right,),
                            device_id_type=pl.DeviceIdType.MESH)
        pl.semaphore_wait(bar, 2)

        xv = x_ref[...]
        conv = xv * k_ref[T - 1, :]
        for t in range(T - 1):
            sh = T - 1 - t
            shifted = jnp.concatenate(
                [jnp.zeros((B, sh, C), jnp.float32), xv[:, :S - sh, :]],
                axis=1)
            conv = conv + shifted * k_ref[t, :]
        a = conv / (1.0 + jnp.exp(-conv))
        p_ref[...] = jnp.dot(a.reshape(R, C), wp_ref[...],
                             preferred_element_type=jnp.float32)

        for s in range(N_DEV):
            row = ((d - s + N_DEV) % N_DEV) * CH
            pr_ref[s, :, :] = p_ref[pl.ds(row, CH), :]

        for h in range(N_DEV - 1):
            rdma = pltpu.make_async_remote_copy(
                src_ref=pr_ref.at[h],
                dst_ref=rs_comm.at[h],
                send_sem=rs_send.at[h],
                recv_sem=rs_recv.at[h],
                device_id=(right,),
                device_id_type=pl.DeviceIdType.MESH,
            )
            rdma.start()
            rdma.wait()
            pr_ref[h + 1, :, :] = pr_ref[h + 1, :, :] + rs_comm[h, :, :]

        ag_ref[0, :, :] = pr_ref[N_DEV - 1, :, :]
        for h in range(N_DEV - 1):
            rdma = pltpu.make_async_remote_copy(
                src_ref=ag_ref.at[h],
                dst_ref=ag_ref.at[h + 1],
                send_sem=ag_send.at[h],
                recv_sem=ag_recv.at[h],
                device_id=(right,),
                device_id_type=pl.DeviceIdType.MESH,
            )
            rdma.start()
            rdma.wait()

        for a_slot in range(N_DEV):
            c = (d + 1 - a_slot + N_DEV) % N_DEV
            out_ref[pl.ds(c * CH, CH), :] = ag_ref[a_slot, :, :]

    out = pl.pallas_call(
        body,
        out_shape=jax.ShapeDtypeStruct((R, P), jnp.float32),
        in_specs=[pl.BlockSpec(memory_space=pltpu.VMEM)] * 3,
        out_specs=pl.BlockSpec(memory_space=pltpu.VMEM),
        scratch_shapes=[
            pltpu.VMEM((R, P), jnp.float32),
            pltpu.VMEM((N_DEV, CH, P), jnp.float32),
            pltpu.VMEM((N_DEV, CH, P), jnp.float32),
            pltpu.VMEM((N_DEV - 1, CH, P), jnp.float32),
            pltpu.SemaphoreType.DMA((N_DEV - 1,)),
            pltpu.SemaphoreType.DMA((N_DEV - 1,)),
            pltpu.SemaphoreType.DMA((N_DEV - 1,)),
            pltpu.SemaphoreType.DMA((N_DEV - 1,)),
        ],
        compiler_params=pltpu.CompilerParams(
            collective_id=0,
            vmem_limit_bytes=100 * 1024 * 1024,
        ),
    )(x, k, Wp)
    return out.reshape(B, S, P)


# device time: 88942 ns/iter; 2.3333x vs baseline; 2.3333x over previous
import jax
import jax.numpy as jnp
from jax import lax
from jax.experimental import pallas as pl
from jax.experimental.pallas import tpu as pltpu

N_DEV = 8

_PARTS = (
    (0, 1344, (1, 3, 4), ("p", "b1", "b2")),
    (1344, 1344, (3, 4, 1), ("b1", "b2", "b0")),
    (2688, 1408, (4, 1, 3), ("b2", "p", "b1")),
)
_SEC = (0, 4, 6)


def kernel(x, k, Wp):
    B, S, C = x.shape
    T = k.shape[0]
    _, P = Wp.shape
    R = B * S

    def body(x_ref, k_ref, wp_ref, out_ref,
             p_ref, rs0, rs1, rs2, send_sems, recv_sems):
        rs_comm = (rs0, rs1, rs2)
        d = lax.axis_index("i")
        b0 = d & 1
        b1 = (d >> 1) & 1
        b2 = (d >> 2) & 1
        sels = {"b0": b0, "b1": b1, "b2": b2, "p": b0 ^ b1}

        bar = pltpu.get_barrier_semaphore()
        for m in (1, 3, 4):
            pl.semaphore_signal(bar, inc=1, device_id=(d ^ m,),
                                device_id_type=pl.DeviceIdType.MESH)
        pl.semaphore_wait(bar, 3)

        xv = x_ref[...]
        conv = xv * k_ref[T - 1, :]
        for t in range(T - 1):
            sh = T - 1 - t
            shifted = jnp.concatenate(
                [jnp.zeros((B, sh, C), jnp.float32), xv[:, :S - sh, :]],
                axis=1)
            conv = conv + shifted * k_ref[t, :]
        a = conv / (1.0 + jnp.exp(-conv))
        p_ref[...] = jnp.dot(a.reshape(R, C), wp_ref[...],
                               preferred_element_type=jnp.float32)

        for kstep in range(3):
            ns = 4 >> kstep
            rdmas = []
            for p, (start, rows, masks, selnames) in enumerate(_PARTS):
                u = rows // 8
                s_bits = [sels[n] for n in selnames]
                base = 0
                for j in range(kstep):
                    base = base + s_bits[j] * (4 >> j)
                send0 = (base + (1 - s_bits[kstep]) * ns) * u + start
                keep0 = (base + s_bits[kstep] * ns) * u + start
                rdma = pltpu.make_async_remote_copy(
                    src_ref=p_ref.at[pl.ds(send0, ns * u)],
                    dst_ref=rs_comm[p].at[pl.ds(_SEC[kstep] * u, ns * u)],
                    send_sem=send_sems.at[p, kstep],
                    recv_sem=recv_sems.at[p, kstep],
                    device_id=(d ^ masks[kstep],),
                    device_id_type=pl.DeviceIdType.MESH,
                )
                rdma.start()
                rdmas.append((rdma, p, keep0, ns * u, _SEC[kstep] * u))
            for rdma, p, keep0, nr, sec in rdmas:
                rdma.wait()
                p_ref[pl.ds(keep0, nr), :] = (
                    p_ref[pl.ds(keep0, nr), :]
                    + rs_comm[p][pl.ds(sec, nr), :]
                )

        for t_idx, kstep in enumerate((2, 1, 0)):
            ns = 4 >> kstep
            sem_t = 3 + t_idx
            rdmas = []
            for p, (start, rows, masks, selnames) in enumerate(_PARTS):
                u = rows // 8
                s_bits = [sels[n] for n in selnames]
                base = 0
                for j in range(kstep):
                    base = base + s_bits[j] * (4 >> j)
                held0 = (base + s_bits[kstep] * ns) * u + start
                rdma = pltpu.make_async_remote_copy(
                    src_ref=p_ref.at[pl.ds(held0, ns * u)],
                    dst_ref=p_ref.at[pl.ds(held0, ns * u)],
                    send_sem=send_sems.at[p, sem_t],
                    recv_sem=recv_sems.at[p, sem_t],
                    device_id=(d ^ masks[kstep],),
                    device_id_type=pl.DeviceIdType.MESH,
                )
                rdma.start()
                rdmas.append(rdma)
            for rdma in rdmas:
                rdma.wait()

        out_ref[...] = p_ref[...]

    out = pl.pallas_call(
        body,
        out_shape=jax.ShapeDtypeStruct((R, P), jnp.float32),
        in_specs=[pl.BlockSpec(memory_space=pltpu.VMEM)] * 3,
        out_specs=pl.BlockSpec(memory_space=pltpu.VMEM),
        scratch_shapes=[
            pltpu.VMEM((R, P), jnp.float32),
            pltpu.VMEM((7 * (1344 // 8), P), jnp.float32),
            pltpu.VMEM((7 * (1344 // 8), P), jnp.float32),
            pltpu.VMEM((7 * (1408 // 8), P), jnp.float32),
            pltpu.SemaphoreType.DMA((3, 6)),
            pltpu.SemaphoreType.DMA((3, 6)),
        ],
        compiler_params=pltpu.CompilerParams(
            collective_id=0,
            vmem_limit_bytes=100 * 1024 * 1024,
        ),
    )(x, k, Wp)
    return out.reshape(B, S, P)
